# baseline (speedup 1.0000x reference)
import functools

import jax
import jax.numpy as jnp
from jax import lax
from jax.experimental import pallas as pl
from jax.experimental.pallas import tpu as pltpu
from jax.experimental.pallas import tpu_sc as plsc

_OMEGA = 64
_SINK = 4
_MAX_WINDOWS = 30000

_HT = 8
_HSC = 16 - _HT
_TPH = 32 // _HSC
_RCH = 16
_NBUF = 2


def _win_mask(s, nw, score_end):
    c = lax.broadcasted_iota(jnp.int32, (s, 128), 0)
    w = lax.broadcasted_iota(jnp.int32, (s, 128), 1)
    m = ((w < nw) & (c >= _SINK) & (c < score_end)
         & ((c - _SINK) // _OMEGA == w))
    return m.astype(jnp.float32)


def _tc_body(nq, nw, score_end, attn_ref, win_ref, acc_ref):
    h = pl.program_id(0)
    q = pl.program_id(1)

    @pl.when(q == 0)
    def _init():
        acc_ref[...] = jnp.zeros_like(acc_ref)

    acc_ref[...] += jnp.sum(attn_ref[0], axis=0, keepdims=True)

    @pl.when(q == nq - 1)
    def _win():
        m = _win_mask(acc_ref.shape[1], nw, score_end)
        win_ref[pl.ds(h, 1), :] = lax.dot_general(
            acc_ref[...], m, (((1,), (0,)), ((), ())),
            preferred_element_type=jnp.float32)


def _tc_reduce(attn):
    _, s, _ = attn.shape
    qc = 256
    nq = s // qc
    nw = (s - _SINK) // _OMEGA
    score_end = _SINK + nw * _OMEGA
    return pl.pallas_call(
        functools.partial(_tc_body, nq, nw, score_end),
        grid=(_HT, nq),
        in_specs=[pl.BlockSpec((1, qc, s), lambda hh, qq: (hh, qq, 0))],
        out_specs=pl.BlockSpec((_HT, 128), lambda hh, qq: (0, 0)),
        out_shape=jax.ShapeDtypeStruct((_HT, 128), jnp.float32),
        scratch_shapes=[pltpu.VMEM((1, s), jnp.float32)],
        compiler_params=pltpu.CompilerParams(
            dimension_semantics=("arbitrary", "arbitrary")),
    )(attn)


def _sc_body(attn_hbm, out_hbm, buf0, buf1, colacc, sem0, sem1):
    s = attn_hbm.shape[1]
    nvec = s // 16
    cid = lax.axis_index("c")
    sid = lax.axis_index("s")
    wid = sid * 2 + cid
    head = _HT + wid // _TPH
    rows_per_part = s // _TPH
    r0 = (wid % _TPH) * rows_per_part
    bufs = (buf0, buf1)
    sems = (sem0, sem1)
    nch = rows_per_part // _RCH
    npair = nch // _NBUF

    def zero(k, carry):
        colacc[pl.ds(k * 16, 16)] = jnp.zeros((16,), jnp.float32)
        return carry

    lax.fori_loop(0, nvec, zero, 0)

    def dma(r, b):
        return pltpu.make_async_copy(
            attn_hbm.at[head, pl.ds(r0 + r * _RCH, _RCH), :],
            bufs[b], sems[b])

    for b in range(_NBUF):
        dma(b, b).start()

    def pair(g, carry):
        for b in range(_NBUF):
            r = g * _NBUF + b
            dma(r, b).wait()
            buf = bufs[b]

            def row(i, c, buf=buf):
                for j in range(nvec):
                    plsc.addupdate(colacc.at[pl.ds(j * 16, 16)],
                                   buf[i, pl.ds(j * 16, 16)])
                return c

            lax.fori_loop(0, _RCH, row, 0)

            @pl.when(g < npair - 1)
            def _next():
                dma(r + _NBUF, b).start()
        return carry

    lax.fori_loop(0, npair, pair, 0)
    dst = pl.multiple_of(wid * s, 8)
    pltpu.sync_copy(colacc, out_hbm.at[pl.ds(dst, s)])


def _sc_reduce(attn):
    _, s, _ = attn.shape
    mesh = plsc.VectorSubcoreMesh(
        core_axis_name="c", subcore_axis_name="s",
        num_cores=2, num_subcores=16)
    run = pl.kernel(
        _sc_body,
        out_type=jax.ShapeDtypeStruct((_HSC * _TPH * s,), jnp.float32),
        mesh=mesh,
        scratch_types=[
            pltpu.VMEM((_RCH, s), jnp.float32),
            pltpu.VMEM((_RCH, s), jnp.float32),
            pltpu.VMEM((s,), jnp.float32),
            pltpu.SemaphoreType.DMA,
            pltpu.SemaphoreType.DMA,
        ],
        compiler_params=pltpu.CompilerParams(use_tc_tiling_on_sc=True),
    )
    return run(attn)


def _asm_body(nh, s, nw, score_end, win_tc_ref, colsum_ref, out_ref):
    cs = jnp.sum(colsum_ref[...].reshape(_HSC, _TPH, s), axis=1)
    m = _win_mask(s, nw, score_end)
    win_sc = lax.dot_general(
        cs, m, (((1,), (0,)), ((), ())), preferred_element_type=jnp.float32)
    win = jnp.concatenate([win_tc_ref[...], win_sc], axis=0)
    col = lax.broadcasted_iota(jnp.int32, (nh, _MAX_WINDOWS), 1)
    nanp = jnp.full((nh, _MAX_WINDOWS - 128), jnp.nan, dtype=jnp.float32)
    winpad = jnp.concatenate([win, nanp], axis=1)
    live = col < nw
    out_ref[0] = jnp.where(live, winpad, jnp.float32(jnp.nan))
    idx_plane = jnp.where(live, col.astype(jnp.float32), jnp.float32(jnp.nan))
    out_ref[1] = idx_plane
    out_ref[2] = idx_plane


def _assemble(win_tc, colsum_sc, nh, s):
    nw = (s - _SINK) // _OMEGA
    score_end = _SINK + nw * _OMEGA
    return pl.pallas_call(
        functools.partial(_asm_body, nh, s, nw, score_end),
        out_shape=jax.ShapeDtypeStruct((3, nh, _MAX_WINDOWS), jnp.float32),
    )(win_tc, colsum_sc)


def kernel(past_key, past_value, attn_score_cache):
    b, h, s, _ = attn_score_cache.shape
    attn = attn_score_cache.reshape(h, s, s)
    colsum_sc = _sc_reduce(attn)
    win_tc = _tc_reduce(attn)
    planes = _assemble(win_tc, colsum_sc, h, s)
    return jnp.transpose(planes, (1, 2, 0))


# speedup vs baseline: 2.3230x; 2.3230x over previous
import functools

import jax
import jax.numpy as jnp
from jax import lax
from jax.experimental import pallas as pl
from jax.experimental.pallas import tpu as pltpu
from jax.experimental.pallas import tpu_sc as plsc

_OMEGA = 64
_SINK = 4
_MAX_WINDOWS = 30000

_HT = 9
_HSC = 16 - _HT
_CSL = 512
_RROW = 256
_RCH = 64
_NVEC = _CSL // 16
_NBUF = 2


def _win_mask(s, nw, score_end):
    c = lax.broadcasted_iota(jnp.int32, (s, 128), 0)
    w = lax.broadcasted_iota(jnp.int32, (s, 128), 1)
    m = ((w < nw) & (c >= _SINK) & (c < score_end)
         & ((c - _SINK) // _OMEGA == w))
    return m.astype(jnp.float32)


def _tc_body(nq, nw, score_end, attn_ref, win_ref, acc_ref):
    h = pl.program_id(0)
    q = pl.program_id(1)

    @pl.when(q == 0)
    def _init():
        acc_ref[...] = jnp.zeros_like(acc_ref)

    acc_ref[...] += jnp.sum(attn_ref[0], axis=0, keepdims=True)

    @pl.when(q == nq - 1)
    def _win():
        m = _win_mask(acc_ref.shape[1], nw, score_end)
        win_ref[pl.ds(h, 1), :] = lax.dot_general(
            acc_ref[...], m, (((1,), (0,)), ((), ())),
            preferred_element_type=jnp.float32)


def _tc_reduce(attn):
    _, s, _ = attn.shape
    qc = 256
    nq = s // qc
    nw = (s - _SINK) // _OMEGA
    score_end = _SINK + nw * _OMEGA
    return pl.pallas_call(
        functools.partial(_tc_body, nq, nw, score_end),
        grid=(_HT, nq),
        in_specs=[pl.BlockSpec((1, qc, s), lambda hh, qq: (hh, qq, 0))],
        out_specs=pl.BlockSpec((_HT, 128), lambda hh, qq: (0, 0)),
        out_shape=jax.ShapeDtypeStruct((_HT, 128), jnp.float32),
        scratch_shapes=[pltpu.VMEM((1, s), jnp.float32)],
        compiler_params=pltpu.CompilerParams(
            dimension_semantics=("arbitrary", "arbitrary")),
    )(attn)


def _sc_body(attn_hbm, out_hbm, buf0, buf1, colbuf, sem0, sem1):
    s = attn_hbm.shape[1]
    cid = lax.axis_index("c")
    sid = lax.axis_index("s")
    wid = sid * 2 + cid
    nsl = s // _CSL
    c0 = (wid % nsl) * _CSL
    r8 = wid // nsl
    cpt = _RROW // _RCH
    nch = _HSC * cpt
    bufs = (buf0, buf1)
    sems = (sem0, sem1)

    def dma(n, b):
        head = _HT + n // cpt
        r0 = r8 * _RROW + (n % cpt) * _RCH
        return pltpu.make_async_copy(
            attn_hbm.at[head, pl.ds(r0, _RCH), pl.ds(c0, _CSL)],
            bufs[b], sems[b])

    for b in range(_NBUF):
        dma(b, b).start()
    accs = (jnp.zeros((16,), jnp.float32),) * _NVEC
    for n in range(nch):
        b = n % _NBUF
        dma(n, b).wait()
        buf = bufs[b]

        def row(i, a, buf=buf):
            return tuple(a[j] + buf[i, pl.ds(j * 16, 16)]
                         for j in range(_NVEC))

        accs = lax.fori_loop(0, _RCH, row, accs)
        if n + _NBUF < nch:
            dma(n + _NBUF, b).start()
        if n % cpt == cpt - 1:
            for j in range(_NVEC):
                colbuf[pl.ds(j * 16, 16)] = accs[j]
            k = n // cpt
            dst = pl.multiple_of(((k * (s // _RROW) + r8) * s + c0), 8)
            pltpu.sync_copy(colbuf, out_hbm.at[pl.ds(dst, _CSL)])
            accs = (jnp.zeros((16,), jnp.float32),) * _NVEC


def _sc_reduce(attn):
    _, s, _ = attn.shape
    mesh = plsc.VectorSubcoreMesh(
        core_axis_name="c", subcore_axis_name="s",
        num_cores=2, num_subcores=16)
    run = pl.kernel(
        _sc_body,
        out_type=jax.ShapeDtypeStruct((_HSC * (s // _RROW) * s,), jnp.float32),
        mesh=mesh,
        scratch_types=[
            pltpu.VMEM((_RCH, _CSL), jnp.float32),
            pltpu.VMEM((_RCH, _CSL), jnp.float32),
            pltpu.VMEM((_CSL,), jnp.float32),
            pltpu.SemaphoreType.DMA,
            pltpu.SemaphoreType.DMA,
        ],
        compiler_params=pltpu.CompilerParams(use_tc_tiling_on_sc=True),
    )
    return run(attn)


def _asm_body(nh, s, nw, score_end, win_tc_ref, colsum_ref, out_ref):
    cs = jnp.sum(colsum_ref[...].reshape(_HSC, s // _RROW, s), axis=1)
    m = _win_mask(s, nw, score_end)
    win_sc = lax.dot_general(
        cs, m, (((1,), (0,)), ((), ())), preferred_element_type=jnp.float32)
    win = jnp.concatenate([win_tc_ref[...], win_sc], axis=0)
    col = lax.broadcasted_iota(jnp.int32, (nh, _MAX_WINDOWS), 1)
    nanp = jnp.full((nh, _MAX_WINDOWS - 128), jnp.nan, dtype=jnp.float32)
    winpad = jnp.concatenate([win, nanp], axis=1)
    live = col < nw
    out_ref[0] = jnp.where(live, winpad, jnp.float32(jnp.nan))
    idx_plane = jnp.where(live, col.astype(jnp.float32), jnp.float32(jnp.nan))
    out_ref[1] = idx_plane
    out_ref[2] = idx_plane


def _assemble(win_tc, colsum_sc, nh, s):
    nw = (s - _SINK) // _OMEGA
    score_end = _SINK + nw * _OMEGA
    return pl.pallas_call(
        functools.partial(_asm_body, nh, s, nw, score_end),
        out_shape=jax.ShapeDtypeStruct((3, nh, _MAX_WINDOWS), jnp.float32),
    )(win_tc, colsum_sc)


def kernel(past_key, past_value, attn_score_cache):
    b, h, s, _ = attn_score_cache.shape
    attn = attn_score_cache.reshape(h, s, s)
    colsum_sc = _sc_reduce(attn)
    win_tc = _tc_reduce(attn)
    planes = _assemble(win_tc, colsum_sc, h, s)
    return jnp.transpose(planes, (1, 2, 0))


# speedup vs baseline: 2.4015x; 1.0338x over previous
import functools

import jax
import jax.numpy as jnp
from jax import lax
from jax.experimental import pallas as pl
from jax.experimental.pallas import tpu as pltpu
from jax.experimental.pallas import tpu_sc as plsc

_OMEGA = 64
_SINK = 4
_MAX_WINDOWS = 30000

_HT = 9
_HSC = 16 - _HT
_CSL = 512
_RROW = 256
_RCH = 64
_NVEC = _CSL // 16
_NBUF = 2


def _win_mask(s, nw, score_end):
    c = lax.broadcasted_iota(jnp.int32, (s, 128), 0)
    w = lax.broadcasted_iota(jnp.int32, (s, 128), 1)
    m = ((w < nw) & (c >= _SINK) & (c < score_end)
         & ((c - _SINK) // _OMEGA == w))
    return m.astype(jnp.float32)


def _tc_body(nq, nw, score_end, attn_ref, win_ref, acc_ref):
    h = pl.program_id(0)
    q = pl.program_id(1)

    @pl.when(q == 0)
    def _init():
        acc_ref[...] = jnp.zeros_like(acc_ref)

    acc_ref[...] += jnp.sum(attn_ref[0], axis=0, keepdims=True)

    @pl.when(q == nq - 1)
    def _win():
        m = _win_mask(acc_ref.shape[1], nw, score_end)
        win_ref[pl.ds(h, 1), :] = lax.dot_general(
            acc_ref[...], m, (((1,), (0,)), ((), ())),
            preferred_element_type=jnp.float32)


def _tc_reduce(attn):
    _, s, _ = attn.shape
    qc = 512
    nq = s // qc
    nw = (s - _SINK) // _OMEGA
    score_end = _SINK + nw * _OMEGA
    return pl.pallas_call(
        functools.partial(_tc_body, nq, nw, score_end),
        grid=(_HT, nq),
        in_specs=[pl.BlockSpec((1, qc, s), lambda hh, qq: (hh, qq, 0))],
        out_specs=pl.BlockSpec((_HT, 128), lambda hh, qq: (0, 0)),
        out_shape=jax.ShapeDtypeStruct((_HT, 128), jnp.float32),
        scratch_shapes=[pltpu.VMEM((1, s), jnp.float32)],
        compiler_params=pltpu.CompilerParams(
            dimension_semantics=("arbitrary", "arbitrary")),
    )(attn)


def _sc_body(attn_hbm, out_hbm, buf0, buf1, colbuf, sem0, sem1):
    s = attn_hbm.shape[1]
    cid = lax.axis_index("c")
    sid = lax.axis_index("s")
    wid = sid * 2 + cid
    nsl = s // _CSL
    c0 = (wid % nsl) * _CSL
    r8 = wid // nsl
    cpt = _RROW // _RCH
    nch = _HSC * cpt
    bufs = (buf0, buf1)
    sems = (sem0, sem1)

    def dma(n, b):
        head = _HT + n // cpt
        r0 = r8 * _RROW + (n % cpt) * _RCH
        return pltpu.make_async_copy(
            attn_hbm.at[head, pl.ds(r0, _RCH), pl.ds(c0, _CSL)],
            bufs[b], sems[b])

    for b in range(_NBUF):
        dma(b, b).start()
    accs = (jnp.zeros((16,), jnp.float32),) * _NVEC
    for n in range(nch):
        b = n % _NBUF
        dma(n, b).wait()
        buf = bufs[b]

        def row(i, a, buf=buf):
            return tuple(a[j] + buf[i, pl.ds(j * 16, 16)]
                         for j in range(_NVEC))

        accs = lax.fori_loop(0, _RCH, row, accs)
        if n + _NBUF < nch:
            dma(n + _NBUF, b).start()
        if n % cpt == cpt - 1:
            for j in range(_NVEC):
                colbuf[pl.ds(j * 16, 16)] = accs[j]
            k = n // cpt
            dst = pl.multiple_of(((k * (s // _RROW) + r8) * s + c0), 8)
            pltpu.sync_copy(colbuf, out_hbm.at[pl.ds(dst, _CSL)])
            accs = (jnp.zeros((16,), jnp.float32),) * _NVEC


def _sc_reduce(attn):
    _, s, _ = attn.shape
    mesh = plsc.VectorSubcoreMesh(
        core_axis_name="c", subcore_axis_name="s",
        num_cores=2, num_subcores=16)
    run = pl.kernel(
        _sc_body,
        out_type=jax.ShapeDtypeStruct((_HSC * (s // _RROW) * s,), jnp.float32),
        mesh=mesh,
        scratch_types=[
            pltpu.VMEM((_RCH, _CSL), jnp.float32),
            pltpu.VMEM((_RCH, _CSL), jnp.float32),
            pltpu.VMEM((_CSL,), jnp.float32),
            pltpu.SemaphoreType.DMA,
            pltpu.SemaphoreType.DMA,
        ],
        compiler_params=pltpu.CompilerParams(use_tc_tiling_on_sc=True),
    )
    return run(attn)


def _asm_body(nh, s, nw, score_end, win_tc_ref, colsum_ref, out_ref):
    cs = jnp.sum(colsum_ref[...].reshape(_HSC, s // _RROW, s), axis=1)
    m = _win_mask(s, nw, score_end)
    win_sc = lax.dot_general(
        cs, m, (((1,), (0,)), ((), ())), preferred_element_type=jnp.float32)
    win = jnp.concatenate([win_tc_ref[...], win_sc], axis=0)
    col = lax.broadcasted_iota(jnp.int32, (nh, _MAX_WINDOWS), 1)
    nanp = jnp.full((nh, _MAX_WINDOWS - 128), jnp.nan, dtype=jnp.float32)
    winpad = jnp.concatenate([win, nanp], axis=1)
    live = col < nw
    out_ref[0] = jnp.where(live, winpad, jnp.float32(jnp.nan))
    idx_plane = jnp.where(live, col.astype(jnp.float32), jnp.float32(jnp.nan))
    out_ref[1] = idx_plane
    out_ref[2] = idx_plane


def _assemble(win_tc, colsum_sc, nh, s):
    nw = (s - _SINK) // _OMEGA
    score_end = _SINK + nw * _OMEGA
    return pl.pallas_call(
        functools.partial(_asm_body, nh, s, nw, score_end),
        out_shape=jax.ShapeDtypeStruct((3, nh, _MAX_WINDOWS), jnp.float32),
    )(win_tc, colsum_sc)


def kernel(past_key, past_value, attn_score_cache):
    b, h, s, _ = attn_score_cache.shape
    attn = attn_score_cache.reshape(h, s, s)
    colsum_sc = _sc_reduce(attn)
    win_tc = _tc_reduce(attn)
    planes = _assemble(win_tc, colsum_sc, h, s)
    return jnp.transpose(planes, (1, 2, 0))


# speedup vs baseline: 2.4209x; 1.0081x over previous
import functools

import jax
import jax.numpy as jnp
from jax import lax
from jax.experimental import pallas as pl
from jax.experimental.pallas import tpu as pltpu
from jax.experimental.pallas import tpu_sc as plsc

_OMEGA = 64
_SINK = 4
_MAX_WINDOWS = 30000

_HT = 9
_HSC = 16 - _HT
_CSL = 512
_RROW = 256
_RCH = 64
_NVEC = _CSL // 16
_NBUF = 2


def _win_mask(s, nw, score_end):
    c = lax.broadcasted_iota(jnp.int32, (s, 128), 0)
    w = lax.broadcasted_iota(jnp.int32, (s, 128), 1)
    m = ((w < nw) & (c >= _SINK) & (c < score_end)
         & ((c - _SINK) // _OMEGA == w))
    return m.astype(jnp.float32)


def _tc_body(nq, nw, score_end, attn_ref, win_ref, acc_ref):
    h = pl.program_id(0)
    q = pl.program_id(1)

    @pl.when(q == 0)
    def _init():
        acc_ref[...] = jnp.zeros_like(acc_ref)

    acc_ref[...] += jnp.sum(attn_ref[0], axis=0, keepdims=True)

    @pl.when(q == nq - 1)
    def _win():
        m = _win_mask(acc_ref.shape[1], nw, score_end)
        win_ref[pl.ds(h, 1), :] = lax.dot_general(
            acc_ref[...], m, (((1,), (0,)), ((), ())),
            preferred_element_type=jnp.float32)


def _tc_reduce(attn):
    _, s, _ = attn.shape
    qc = 1024
    nq = s // qc
    nw = (s - _SINK) // _OMEGA
    score_end = _SINK + nw * _OMEGA
    return pl.pallas_call(
        functools.partial(_tc_body, nq, nw, score_end),
        grid=(_HT, nq),
        in_specs=[pl.BlockSpec((1, qc, s), lambda hh, qq: (hh, qq, 0))],
        out_specs=pl.BlockSpec((_HT, 128), lambda hh, qq: (0, 0)),
        out_shape=jax.ShapeDtypeStruct((_HT, 128), jnp.float32),
        scratch_shapes=[pltpu.VMEM((1, s), jnp.float32)],
        compiler_params=pltpu.CompilerParams(
            dimension_semantics=("arbitrary", "arbitrary")),
    )(attn)


def _sc_body(attn_hbm, out_hbm, buf0, buf1, colbuf, sem0, sem1):
    s = attn_hbm.shape[1]
    cid = lax.axis_index("c")
    sid = lax.axis_index("s")
    wid = sid * 2 + cid
    nsl = s // _CSL
    c0 = (wid % nsl) * _CSL
    r8 = wid // nsl
    cpt = _RROW // _RCH
    nch = _HSC * cpt
    bufs = (buf0, buf1)
    sems = (sem0, sem1)

    def dma(n, b):
        head = _HT + n // cpt
        r0 = r8 * _RROW + (n % cpt) * _RCH
        return pltpu.make_async_copy(
            attn_hbm.at[head, pl.ds(r0, _RCH), pl.ds(c0, _CSL)],
            bufs[b], sems[b])

    for b in range(_NBUF):
        dma(b, b).start()
    accs = (jnp.zeros((16,), jnp.float32),) * _NVEC
    for n in range(nch):
        b = n % _NBUF
        dma(n, b).wait()
        buf = bufs[b]

        def row(i, a, buf=buf):
            return tuple(a[j] + buf[i, pl.ds(j * 16, 16)]
                         for j in range(_NVEC))

        accs = lax.fori_loop(0, _RCH, row, accs)
        if n + _NBUF < nch:
            dma(n + _NBUF, b).start()
        if n % cpt == cpt - 1:
            for j in range(_NVEC):
                colbuf[pl.ds(j * 16, 16)] = accs[j]
            k = n // cpt
            dst = pl.multiple_of(((k * (s // _RROW) + r8) * s + c0), 8)
            pltpu.sync_copy(colbuf, out_hbm.at[pl.ds(dst, _CSL)])
            accs = (jnp.zeros((16,), jnp.float32),) * _NVEC


def _sc_reduce(attn):
    _, s, _ = attn.shape
    mesh = plsc.VectorSubcoreMesh(
        core_axis_name="c", subcore_axis_name="s",
        num_cores=2, num_subcores=16)
    run = pl.kernel(
        _sc_body,
        out_type=jax.ShapeDtypeStruct((_HSC * (s // _RROW) * s,), jnp.float32),
        mesh=mesh,
        scratch_types=[
            pltpu.VMEM((_RCH, _CSL), jnp.float32),
            pltpu.VMEM((_RCH, _CSL), jnp.float32),
            pltpu.VMEM((_CSL,), jnp.float32),
            pltpu.SemaphoreType.DMA,
            pltpu.SemaphoreType.DMA,
        ],
        compiler_params=pltpu.CompilerParams(use_tc_tiling_on_sc=True),
    )
    return run(attn)


def _asm_body(nh, s, nw, score_end, win_tc_ref, colsum_ref, out_ref):
    cs = jnp.sum(colsum_ref[...].reshape(_HSC, s // _RROW, s), axis=1)
    m = _win_mask(s, nw, score_end)
    win_sc = lax.dot_general(
        cs, m, (((1,), (0,)), ((), ())), preferred_element_type=jnp.float32)
    win = jnp.concatenate([win_tc_ref[...], win_sc], axis=0)
    col = lax.broadcasted_iota(jnp.int32, (nh, _MAX_WINDOWS), 1)
    nanp = jnp.full((nh, _MAX_WINDOWS - 128), jnp.nan, dtype=jnp.float32)
    winpad = jnp.concatenate([win, nanp], axis=1)
    live = col < nw
    out_ref[0] = jnp.where(live, winpad, jnp.float32(jnp.nan))
    idx_plane = jnp.where(live, col.astype(jnp.float32), jnp.float32(jnp.nan))
    out_ref[1] = idx_plane
    out_ref[2] = idx_plane


def _assemble(win_tc, colsum_sc, nh, s):
    nw = (s - _SINK) // _OMEGA
    score_end = _SINK + nw * _OMEGA
    return pl.pallas_call(
        functools.partial(_asm_body, nh, s, nw, score_end),
        out_shape=jax.ShapeDtypeStruct((3, nh, _MAX_WINDOWS), jnp.float32),
    )(win_tc, colsum_sc)


def kernel(past_key, past_value, attn_score_cache):
    b, h, s, _ = attn_score_cache.shape
    attn = attn_score_cache.reshape(h, s, s)
    colsum_sc = _sc_reduce(attn)
    win_tc = _tc_reduce(attn)
    planes = _assemble(win_tc, colsum_sc, h, s)
    return jnp.transpose(planes, (1, 2, 0))
